# Initial kernel scaffold; baseline (speedup 1.0000x reference)
import jax
import jax.numpy as jnp
from jax.experimental import pallas as pl


def kernel(x, edge_index):
    raise NotImplementedError("write your pallas kernel here")


# speedup vs baseline: 5.5436x; 5.5436x over previous
import functools

import jax
import jax.numpy as jnp
from jax import lax
from jax.experimental import pallas as pl
from jax.experimental.pallas import tpu as pltpu
from jax.experimental.pallas import tpu_sc as plsc

NC = 2
NS = 16
NW = NC * NS
LANES = 16


@functools.lru_cache(maxsize=None)
def _build(num_nodes, feat, num_edges):
    per_tile = -(-num_edges // (NW * LANES)) * LANES
    pad_edges = per_tile * NW
    hch = -(-(num_nodes + 1) // (NS * LANES)) * LANES
    hist = hch * NS
    tail = num_nodes - (NS - 1) * hch

    mesh = plsc.VectorSubcoreMesh(
        core_axis_name="c", subcore_axis_name="s", num_cores=NC, num_subcores=NS
    )

    @functools.partial(
        pl.kernel,
        out_type=jax.ShapeDtypeStruct((NC * hist,), jnp.float32),
        mesh=mesh,
        scratch_types=[
            pltpu.VMEM((per_tile,), jnp.int32),
            pltpu.VMEM((per_tile,), jnp.float32),
            pltpu.VMEM((hch,), jnp.float32),
            pltpu.VMEM_SHARED((hist,), jnp.float32),
        ],
    )
    def sc_hist(rows_hbm, out_hbm, idx_v, ones_v, zer_v, hist_s):
        c = lax.axis_index("c")
        s = lax.axis_index("s")
        w = s * NC + c
        one16 = jnp.full((LANES,), 1.0, jnp.float32)
        zero16 = jnp.zeros((LANES,), jnp.float32)

        def fill_ones(i, carry):
            ones_v[pl.ds(i * LANES, LANES)] = one16
            return carry

        lax.fori_loop(0, per_tile // LANES, fill_ones, 0)

        def fill_zeros(i, carry):
            zer_v[pl.ds(i * LANES, LANES)] = zero16
            return carry

        lax.fori_loop(0, hch // LANES, fill_zeros, 0)

        pltpu.sync_copy(zer_v, hist_s.at[pl.ds(s * hch, hch)])
        pltpu.sync_copy(rows_hbm.at[pl.ds(w * per_tile, per_tile)], idx_v)
        plsc.subcore_barrier()
        pltpu.sync_copy(ones_v, hist_s.at[idx_v], add=True)
        plsc.subcore_barrier()
        pltpu.sync_copy(hist_s.at[pl.ds(s * hch, hch)], zer_v)
        pltpu.sync_copy(zer_v, out_hbm.at[pl.ds(c * hist + s * hch, hch)])

    @functools.partial(
        pl.kernel,
        out_type=jax.ShapeDtypeStruct((num_nodes,), jnp.float32),
        mesh=mesh,
        scratch_types=[
            pltpu.VMEM((hch,), jnp.float32),
            pltpu.VMEM((hch,), jnp.float32),
            pltpu.VMEM((hch,), jnp.float32),
            pltpu.VMEM((LANES,), jnp.float32),
            pltpu.VMEM((NS * LANES,), jnp.float32),
            pltpu.VMEM_SHARED((NS * LANES,), jnp.float32),
        ],
    )
    def sc_norm(p_hbm, deg_hbm, v0, v1, dv, mv, am, maxs_s):
        s = lax.axis_index("s")
        pltpu.sync_copy(p_hbm.at[pl.ds(s * hch, hch)], v0)
        pltpu.sync_copy(p_hbm.at[pl.ds(hist + s * hch, hch)], v1)
        lanes = lax.broadcasted_iota(jnp.int32, (LANES,), 0)

        def comb(i, mx):
            v = v0[pl.ds(i * LANES, LANES)] + v1[pl.ds(i * LANES, LANES)]
            dv[pl.ds(i * LANES, LANES)] = v
            gidx = s * hch + i * LANES + lanes
            return jnp.maximum(mx, jnp.where(gidx < num_nodes, v, 0.0))

        mx = lax.fori_loop(0, hch // LANES, comb, jnp.zeros((LANES,), jnp.float32))
        mv[pl.ds(0, LANES)] = mx
        pltpu.sync_copy(mv, maxs_s.at[pl.ds(s * LANES, LANES)])
        plsc.subcore_barrier()
        pltpu.sync_copy(maxs_s, am)

        def mcomb(j, m2):
            return jnp.maximum(m2, am[pl.ds(j * LANES, LANES)])

        mx2 = lax.fori_loop(0, NS, mcomb, jnp.zeros((LANES,), jnp.float32))
        dnums = lax.GatherDimensionNumbers(
            offset_dims=(), collapsed_slice_dims=(0,), start_index_map=(0,)
        )
        for shift in (1, 2, 4, 8):
            perm = lanes ^ shift
            shuf = lax.gather(
                mx2,
                perm[:, None],
                dnums,
                slice_sizes=(1,),
                mode=lax.GatherScatterMode.PROMISE_IN_BOUNDS,
            )
            mx2 = jnp.maximum(mx2, shuf)
        inv = 1.0 / mx2

        def scale(i, carry):
            dv[pl.ds(i * LANES, LANES)] = dv[pl.ds(i * LANES, LANES)] * inv
            return carry

        lax.fori_loop(0, hch // LANES, scale, 0)

        @pl.when(s < NS - 1)
        def _():
            pltpu.sync_copy(dv, deg_hbm.at[pl.ds(s * hch, hch)])

        @pl.when(s == NS - 1)
        def _():
            pltpu.sync_copy(
                dv.at[pl.ds(0, tail)], deg_hbm.at[pl.ds((NS - 1) * hch, tail)]
            )

    rb = 1000

    def cat_body(x_ref, d_ref, o_ref):
        o_ref[:, :feat] = x_ref[...]
        o_ref[:, feat : feat + 1] = d_ref[...]

    tc_concat = pl.pallas_call(
        cat_body,
        grid=(num_nodes // rb,),
        in_specs=[
            pl.BlockSpec((rb, feat), lambda i: (i, 0)),
            pl.BlockSpec((rb, 1), lambda i: (i, 0)),
        ],
        out_specs=pl.BlockSpec((rb, feat + 1), lambda i: (i, 0)),
        out_shape=jax.ShapeDtypeStruct((num_nodes, feat + 1), jnp.float32),
    )

    def run(x, edge_index):
        row = edge_index[0].astype(jnp.int32)
        pad = jnp.full((pad_edges - num_edges,), num_nodes, jnp.int32)
        rows = jnp.concatenate([row, pad])
        partials = sc_hist(rows)
        deg = sc_norm(partials)
        return tc_concat(x, deg.reshape(num_nodes, 1))

    return run


def kernel(x, edge_index):
    return _build(x.shape[0], x.shape[1], edge_index.shape[1])(x, edge_index)
